# Initial kernel scaffold; baseline (speedup 1.0000x reference)
import jax
import jax.numpy as jnp
from jax.experimental import pallas as pl


def kernel(ss_matrix, embed_table, ln_gamma, ln_beta, W):
    raise NotImplementedError("write your pallas kernel here")


# speedup vs baseline: 21.7446x; 21.7446x over previous
import functools

import jax
import jax.numpy as jnp
from jax import lax
from jax.experimental import pallas as pl

N_CLASSES = 4
C_Z = 128
EPS = 1e-5
BLK = 8192


def _encode_block(ids_ref, embed_ref, gamma_ref, beta_ref, w_ref, out_ref):
    e = embed_ref[:, :]
    mean = jnp.mean(e, axis=1, keepdims=True)
    var = jnp.mean(jnp.square(e - mean), axis=1, keepdims=True)
    norm = (e - mean) * lax.rsqrt(var + EPS) * gamma_ref[:, :] + beta_ref[:, :]
    table4 = lax.dot_general(norm, w_ref[:, :], (((1,), (1,)), ((), ())),
                             preferred_element_type=jnp.float32)
    ids = ids_ref[0, 0, :]
    classes = lax.broadcasted_iota(jnp.int32, (BLK, N_CLASSES), 1)
    onehot = (ids[:, None] == classes).astype(jnp.float32)
    out_ref[:, :] = lax.dot_general(onehot, table4, (((1,), (0,)), ((), ())),
                                    preferred_element_type=jnp.float32)


@jax.jit
def kernel(ss_matrix, embed_table, ln_gamma, ln_beta, W):
    b, n, _ = ss_matrix.shape
    total = b * n * n
    num_blocks = total // BLK
    ids = ss_matrix.reshape(num_blocks, 1, BLK)
    gamma2 = ln_gamma.reshape(1, C_Z)
    beta2 = ln_beta.reshape(1, C_Z)
    out = pl.pallas_call(
        _encode_block,
        grid=(num_blocks,),
        in_specs=[
            pl.BlockSpec((1, 1, BLK), lambda i: (i, 0, 0)),
            pl.BlockSpec((N_CLASSES, C_Z), lambda i: (0, 0)),
            pl.BlockSpec((1, C_Z), lambda i: (0, 0)),
            pl.BlockSpec((1, C_Z), lambda i: (0, 0)),
            pl.BlockSpec((C_Z, C_Z), lambda i: (0, 0)),
        ],
        out_specs=pl.BlockSpec((BLK, C_Z), lambda i: (i, 0)),
        out_shape=jax.ShapeDtypeStruct((total, C_Z), jnp.float32),
    )(ids, embed_table, gamma2, beta2, W)
    return out.reshape(b, n, n, C_Z)


# speedup vs baseline: 23.5711x; 1.0840x over previous
import functools

import jax
import jax.numpy as jnp
from jax import lax
from jax.experimental import pallas as pl

N_CLASSES = 4
C_Z = 128
EPS = 1e-5
BLK = 8192


def _encode_block(ids_ref, embed_ref, gamma_ref, beta_ref, w_ref, out_ref):
    e = embed_ref[:, :]
    mean = jnp.mean(e, axis=1, keepdims=True)
    var = jnp.mean(jnp.square(e - mean), axis=1, keepdims=True)
    norm = (e - mean) * lax.rsqrt(var + EPS) * gamma_ref[:, :] + beta_ref[:, :]
    table4 = lax.dot_general(norm, w_ref[:, :], (((1,), (1,)), ((), ())),
                             preferred_element_type=jnp.float32)
    ids = ids_ref[0, :, :]
    classes = lax.broadcasted_iota(jnp.int32, (N_CLASSES, BLK), 0)
    onehot_t = (ids == classes).astype(jnp.float32)
    out_ref[:, :] = lax.dot_general(onehot_t, table4, (((0,), (0,)), ((), ())),
                                    preferred_element_type=jnp.float32)


@jax.jit
def kernel(ss_matrix, embed_table, ln_gamma, ln_beta, W):
    b, n, _ = ss_matrix.shape
    total = b * n * n
    num_blocks = total // BLK
    ids = ss_matrix.reshape(num_blocks, 1, BLK)
    gamma2 = ln_gamma.reshape(1, C_Z)
    beta2 = ln_beta.reshape(1, C_Z)
    out = pl.pallas_call(
        _encode_block,
        grid=(num_blocks,),
        in_specs=[
            pl.BlockSpec((1, 1, BLK), lambda i: (i, 0, 0)),
            pl.BlockSpec((N_CLASSES, C_Z), lambda i: (0, 0)),
            pl.BlockSpec((1, C_Z), lambda i: (0, 0)),
            pl.BlockSpec((1, C_Z), lambda i: (0, 0)),
            pl.BlockSpec((C_Z, C_Z), lambda i: (0, 0)),
        ],
        out_specs=pl.BlockSpec((BLK, C_Z), lambda i: (i, 0)),
        out_shape=jax.ShapeDtypeStruct((total, C_Z), jnp.float32),
    )(ids, embed_table, gamma2, beta2, W)
    return out.reshape(b, n, n, C_Z)


# speedup vs baseline: 26.8473x; 1.1390x over previous
import functools

import jax
import jax.numpy as jnp
from jax import lax
from jax.experimental import pallas as pl

N_CLASSES = 4
C_Z = 128
EPS = 1e-5
BLK = 16384


def _encode_block(ids_ref, embed_ref, gamma_ref, beta_ref, w_ref, out_ref):
    e = embed_ref[:, :]
    mean = jnp.mean(e, axis=1, keepdims=True)
    var = jnp.mean(jnp.square(e - mean), axis=1, keepdims=True)
    norm = (e - mean) * lax.rsqrt(var + EPS) * gamma_ref[:, :] + beta_ref[:, :]
    table4 = lax.dot_general(norm, w_ref[:, :], (((1,), (1,)), ((), ())),
                             preferred_element_type=jnp.float32)
    ids = ids_ref[0, :, :]
    classes = lax.broadcasted_iota(jnp.int32, (N_CLASSES, BLK), 0)
    onehot_t = (ids == classes).astype(jnp.float32)
    out_ref[:, :] = lax.dot_general(onehot_t, table4, (((0,), (0,)), ((), ())),
                                    preferred_element_type=jnp.float32)


@jax.jit
def kernel(ss_matrix, embed_table, ln_gamma, ln_beta, W):
    b, n, _ = ss_matrix.shape
    total = b * n * n
    num_blocks = total // BLK
    ids = ss_matrix.reshape(num_blocks, 1, BLK)
    gamma2 = ln_gamma.reshape(1, C_Z)
    beta2 = ln_beta.reshape(1, C_Z)
    out = pl.pallas_call(
        _encode_block,
        grid=(num_blocks,),
        in_specs=[
            pl.BlockSpec((1, 1, BLK), lambda i: (i, 0, 0)),
            pl.BlockSpec((N_CLASSES, C_Z), lambda i: (0, 0)),
            pl.BlockSpec((1, C_Z), lambda i: (0, 0)),
            pl.BlockSpec((1, C_Z), lambda i: (0, 0)),
            pl.BlockSpec((C_Z, C_Z), lambda i: (0, 0)),
        ],
        out_specs=pl.BlockSpec((BLK, C_Z), lambda i: (i, 0)),
        out_shape=jax.ShapeDtypeStruct((total, C_Z), jnp.float32),
    )(ids, embed_table, gamma2, beta2, W)
    return out.reshape(b, n, n, C_Z)
